# Initial kernel scaffold; baseline (speedup 1.0000x reference)
import functools

import jax
import jax.numpy as jnp
from jax import lax
from jax.experimental import pallas as pl
from jax.experimental.pallas import tpu as pltpu
from jax.experimental.pallas import tpu_sc as plsc

N = 10000
D = 256
T = 160000
MARGIN = 1.0

NC = 2
NS = 16
NW = NC * NS
TPW = T // NW
CHUNK = 40
NCHUNK = TPW // CHUNK
NLANE = 16
NSEG = D // NLANE


def _sc_kernel(h1, h2, h3, ai, pi, ni, out,
               ai_v, pi_v, ni_v, av, pv, nv, acc_v, sem):
    cid = lax.axis_index("c")
    sid = lax.axis_index("s")
    wid = sid * NC + cid
    base = wid * TPW

    pltpu.sync_copy(ai.at[pl.ds(base, TPW)], ai_v)
    pltpu.sync_copy(pi.at[pl.ds(base, TPW)], pi_v)
    pltpu.sync_copy(ni.at[pl.ds(base, TPW)], ni_v)

    def chunk_body(g, acc):
        off = g * CHUNK
        cp1 = pltpu.async_copy(h1.at[ai_v.at[pl.ds(off, CHUNK)]], av, sem)
        cp2 = pltpu.async_copy(h2.at[pi_v.at[pl.ds(off, CHUNK)]], pv, sem)
        cp3 = pltpu.async_copy(h3.at[ni_v.at[pl.ds(off, CHUNK)]], nv, sem)
        cp1.wait()
        cp2.wait()
        cp3.wait()

        def t_body(t, acc2):
            part = jnp.zeros((NLANE,), jnp.float32)
            for j in range(NSEG):
                sl = pl.ds(j * NLANE, NLANE)
                a = av[t, sl]
                p = pv[t, sl]
                q = nv[t, sl]
                d1 = a - p
                d2 = a - q
                part = part + (d1 * d1 - d2 * d2)
            c = jnp.sum(part)
            return acc2 + jnp.maximum(c + MARGIN, 0.0)

        return lax.fori_loop(0, CHUNK, t_body, acc)

    total = lax.fori_loop(0, NCHUNK, chunk_body, jnp.float32(0.0))
    acc_v[...] = jnp.full((NLANE,), total, jnp.float32)
    pltpu.sync_copy(acc_v, out.at[wid])


@jax.jit
def kernel(h_c1, h_c2, h_c3, triplets):
    tr = triplets.astype(jnp.int32)
    ai = jnp.ascontiguousarray(tr[:, 0])
    pi = jnp.ascontiguousarray(tr[:, 1])
    ni = jnp.ascontiguousarray(tr[:, 2])

    mesh = plsc.VectorSubcoreMesh(core_axis_name="c", subcore_axis_name="s")
    run = pl.kernel(
        _sc_kernel,
        out_type=jax.ShapeDtypeStruct((NW, NLANE), jnp.float32),
        mesh=mesh,
        scratch_types=[
            pltpu.VMEM((TPW,), jnp.int32),
            pltpu.VMEM((TPW,), jnp.int32),
            pltpu.VMEM((TPW,), jnp.int32),
            pltpu.VMEM((CHUNK, D), jnp.float32),
            pltpu.VMEM((CHUNK, D), jnp.float32),
            pltpu.VMEM((CHUNK, D), jnp.float32),
            pltpu.VMEM((NLANE,), jnp.float32),
            pltpu.SemaphoreType.DMA,
        ],
    )
    partials = run(h_c1, h_c2, h_c3, ai, pi, ni)
    total = jnp.sum(partials) / NLANE
    return total / T + 1e-16


# speedup vs baseline: 3.1511x; 3.1511x over previous
import functools

import jax
import jax.numpy as jnp
from jax import lax
from jax.experimental import pallas as pl
from jax.experimental.pallas import tpu as pltpu
from jax.experimental.pallas import tpu_sc as plsc

N = 10000
D = 256
T = 160000
MARGIN = 1.0

NC = 2
NS = 16
NW = NC * NS
TPW = T // NW
CHUNK = 40
NCHUNK = TPW // CHUNK
NLANE = 16
NSEG = D // NLANE


def _sc_kernel(h1, h2, h3, ai, pi, ni, out,
               ai_v, pi_v, ni_v, av, pv, nv, acc_v, sem):
    cid = lax.axis_index("c")
    sid = lax.axis_index("s")
    wid = sid * NC + cid
    base = wid * TPW

    pltpu.sync_copy(ai.at[pl.ds(base, TPW)], ai_v)
    pltpu.sync_copy(pi.at[pl.ds(base, TPW)], pi_v)
    pltpu.sync_copy(ni.at[pl.ds(base, TPW)], ni_v)

    def chunk_body(g, acc):
        off = g * CHUNK
        cp1 = pltpu.async_copy(h1.at[ai_v.at[pl.ds(off, CHUNK)]], av, sem)
        cp2 = pltpu.async_copy(h2.at[pi_v.at[pl.ds(off, CHUNK)]], pv, sem)
        cp3 = pltpu.async_copy(h3.at[ni_v.at[pl.ds(off, CHUNK)]], nv, sem)
        cp1.wait()
        cp2.wait()
        cp3.wait()

        def t_body(t, acc2):
            part = jnp.zeros((NLANE,), jnp.float32)
            for j in range(NSEG):
                sl = pl.ds(j * NLANE, NLANE)
                a = av[t, sl]
                p = pv[t, sl]
                q = nv[t, sl]
                d1 = a - p
                d2 = a - q
                part = part + (d1 * d1 - d2 * d2)
            c = jnp.sum(part)
            return acc2 + jnp.maximum(c + MARGIN, 0.0)

        return lax.fori_loop(0, CHUNK, t_body, acc)

    total = lax.fori_loop(0, NCHUNK, chunk_body, jnp.float32(0.0))
    acc_v[...] = jnp.full((NLANE,), total, jnp.float32)
    pltpu.sync_copy(acc_v, out.at[wid])


@jax.jit
def kernel(h_c1, h_c2, h_c3, triplets):
    tr = triplets.astype(jnp.int32)
    ai = tr[:, 0]
    pi = tr[:, 1]
    ni = tr[:, 2]

    mesh = plsc.VectorSubcoreMesh(core_axis_name="c", subcore_axis_name="s", num_cores=NC, num_subcores=NS)
    run = pl.kernel(
        _sc_kernel,
        out_type=jax.ShapeDtypeStruct((NW, NLANE), jnp.float32),
        mesh=mesh,
        compiler_params=pltpu.CompilerParams(needs_layout_passes=False),
        scratch_types=[
            pltpu.VMEM((TPW,), jnp.int32),
            pltpu.VMEM((TPW,), jnp.int32),
            pltpu.VMEM((TPW,), jnp.int32),
            pltpu.VMEM((CHUNK, D), jnp.float32),
            pltpu.VMEM((CHUNK, D), jnp.float32),
            pltpu.VMEM((CHUNK, D), jnp.float32),
            pltpu.VMEM((NLANE,), jnp.float32),
            pltpu.SemaphoreType.DMA,
        ],
    )
    partials = run(h_c1, h_c2, h_c3, ai, pi, ni)
    total = jnp.sum(partials) / NLANE
    return total / T + 1e-16


# speedup vs baseline: 5.5941x; 1.7753x over previous
import jax
import jax.numpy as jnp
from jax import lax
from jax.experimental import pallas as pl
from jax.experimental.pallas import tpu as pltpu
from jax.experimental.pallas import tpu_sc as plsc

N = 10000
D = 256
T = 160000
MARGIN = 1.0

NC = 2
NS = 16
NW = NC * NS
TPW = T // NW
CHUNK = 40
NCHUNK = TPW // CHUNK
NPAIR = (NCHUNK - 1) // 2
NLANE = 16
NSEG = D // NLANE


def _sc_kernel(h1, h2, h3, ai, pi, ni, out,
               ai_v, pi_v, ni_v,
               av0, pv0, nv0, av1, pv1, nv1, acc_v, sem0, sem1):
    cid = lax.axis_index("c")
    sid = lax.axis_index("s")
    wid = sid * NC + cid
    base = wid * TPW

    pltpu.sync_copy(ai.at[pl.ds(base, TPW)], ai_v)
    pltpu.sync_copy(pi.at[pl.ds(base, TPW)], pi_v)
    pltpu.sync_copy(ni.at[pl.ds(base, TPW)], ni_v)

    def copies(g, bufs, sem):
        off = g * CHUNK
        ba, bp, bn = bufs
        return (
            pltpu.make_async_copy(h1.at[ai_v.at[pl.ds(off, CHUNK)]], ba, sem),
            pltpu.make_async_copy(h2.at[pi_v.at[pl.ds(off, CHUNK)]], bp, sem),
            pltpu.make_async_copy(h3.at[ni_v.at[pl.ds(off, CHUNK)]], bn, sem),
        )

    def start(g, bufs, sem):
        for cp in copies(g, bufs, sem):
            cp.start()

    def wait(g, bufs, sem):
        for cp in copies(g, bufs, sem):
            cp.wait()

    def compute(bufs, acc):
        ba, bp, bn = bufs

        def t_body(t, acc2):
            part = jnp.zeros((NLANE,), jnp.float32)
            for j in range(NSEG):
                sl = pl.ds(j * NLANE, NLANE)
                a = ba[t, sl]
                p = bp[t, sl]
                q = bn[t, sl]
                d1 = a - p
                d2 = a - q
                part = part + (d1 * d1 - d2 * d2)
            c = jnp.sum(part)
            return acc2 + jnp.maximum(c + MARGIN, 0.0)

        return lax.fori_loop(0, CHUNK, t_body, acc)

    bufsA = (av0, pv0, nv0)
    bufsB = (av1, pv1, nv1)

    start(0, bufsA, sem0)

    def pair_body(i, acc):
        g = 2 * i
        start(g + 1, bufsB, sem1)
        wait(g, bufsA, sem0)
        acc = compute(bufsA, acc)
        start(g + 2, bufsA, sem0)
        wait(g + 1, bufsB, sem1)
        return compute(bufsB, acc)

    acc = lax.fori_loop(0, NPAIR, pair_body, jnp.float32(0.0))
    wait(NCHUNK - 1, bufsA, sem0)
    total = compute(bufsA, acc)

    acc_v[...] = jnp.full((NLANE,), total, jnp.float32)
    pltpu.sync_copy(acc_v, out.at[wid])


@jax.jit
def kernel(h_c1, h_c2, h_c3, triplets):
    tr = triplets.astype(jnp.int32)
    ai = tr[:, 0]
    pi = tr[:, 1]
    ni = tr[:, 2]

    mesh = plsc.VectorSubcoreMesh(core_axis_name="c", subcore_axis_name="s",
                                  num_cores=NC, num_subcores=NS)
    run = pl.kernel(
        _sc_kernel,
        out_type=jax.ShapeDtypeStruct((NW, NLANE), jnp.float32),
        mesh=mesh,
        compiler_params=pltpu.CompilerParams(needs_layout_passes=False),
        scratch_types=[
            pltpu.VMEM((TPW,), jnp.int32),
            pltpu.VMEM((TPW,), jnp.int32),
            pltpu.VMEM((TPW,), jnp.int32),
            pltpu.VMEM((CHUNK, D), jnp.float32),
            pltpu.VMEM((CHUNK, D), jnp.float32),
            pltpu.VMEM((CHUNK, D), jnp.float32),
            pltpu.VMEM((CHUNK, D), jnp.float32),
            pltpu.VMEM((CHUNK, D), jnp.float32),
            pltpu.VMEM((CHUNK, D), jnp.float32),
            pltpu.VMEM((NLANE,), jnp.float32),
            pltpu.SemaphoreType.DMA,
            pltpu.SemaphoreType.DMA,
        ],
    )
    partials = run(h_c1, h_c2, h_c3, ai, pi, ni)
    total = jnp.sum(partials) / NLANE
    return total / T + 1e-16
